# Initial kernel scaffold; baseline (speedup 1.0000x reference)
import jax
import jax.numpy as jnp
from jax.experimental import pallas as pl


def kernel(token_ids, weight):
    raise NotImplementedError("write your pallas kernel here")


# speedup vs baseline: 2.9669x; 2.9669x over previous
import functools

import jax
import jax.numpy as jnp
from jax import lax
from jax.experimental import pallas as pl
from jax.experimental.pallas import tpu as pltpu
from jax.experimental.pallas import tpu_sc as plsc

EMB_DIM = 128
CHUNK = 128


@functools.cache
def _build(num_tokens: int):
    info = plsc.get_sparse_core_info()
    nc, ns = info.num_cores, info.num_subcores
    nw = nc * ns
    assert num_tokens % (nw * CHUNK) == 0
    n_chunks = num_tokens // (nw * CHUNK)
    mesh = plsc.VectorSubcoreMesh(core_axis_name="c", subcore_axis_name="s")

    @functools.partial(
        pl.kernel,
        out_type=jax.ShapeDtypeStruct((nw, n_chunks, CHUNK, EMB_DIM), jnp.float32),
        mesh=mesh,
        scratch_types=[
            pltpu.VMEM((n_chunks, CHUNK), jnp.int32),
            pltpu.VMEM((CHUNK, EMB_DIM), jnp.float32),
            pltpu.SemaphoreType.DMA,
        ],
    )
    def embed(idx_hbm, table_hbm, out_hbm, idx_v, rows_v, gsem):
        wid = lax.axis_index("s") * nc + lax.axis_index("c")
        pltpu.sync_copy(idx_hbm.at[wid], idx_v)

        def chunk_body(c, carry):
            pltpu.async_copy(table_hbm.at[idx_v.at[c]], rows_v, gsem).wait()
            pltpu.sync_copy(rows_v, out_hbm.at[wid, c])
            return carry

        lax.fori_loop(0, n_chunks, chunk_body, 0)

    return embed, nw, n_chunks


def kernel(token_ids, weight):
    b, s = token_ids.shape
    num_tokens = b * s
    embed, nw, n_chunks = _build(num_tokens)
    ids = token_ids.reshape(nw, n_chunks, CHUNK).astype(jnp.int32)
    out = embed(ids, weight)
    return out.reshape(b, s, EMB_DIM)


# speedup vs baseline: 3.3140x; 1.1170x over previous
import functools

import jax
import jax.numpy as jnp
from jax import lax
from jax.experimental import pallas as pl
from jax.experimental.pallas import tpu as pltpu
from jax.experimental.pallas import tpu_sc as plsc

EMB_DIM = 128
CHUNK = 100
NBUF = 4
LA = 2


@functools.cache
def _build(num_tokens: int):
    info = plsc.get_sparse_core_info()
    nc, ns = info.num_cores, info.num_subcores
    nw = nc * ns
    assert num_tokens % (nw * CHUNK) == 0
    n_chunks = num_tokens // (nw * CHUNK)
    assert n_chunks % NBUF == 0 and LA < NBUF
    mesh = plsc.VectorSubcoreMesh(core_axis_name="c", subcore_axis_name="s")

    @functools.partial(
        pl.kernel,
        out_type=jax.ShapeDtypeStruct((nw, n_chunks, CHUNK, EMB_DIM), jnp.float32),
        mesh=mesh,
        scratch_types=[
            pltpu.VMEM((n_chunks, CHUNK), jnp.int32),
            pltpu.VMEM((NBUF, CHUNK, EMB_DIM), jnp.float32),
        ]
        + [pltpu.SemaphoreType.DMA] * (2 * NBUF),
    )
    def embed(idx_hbm, table_hbm, out_hbm, idx_v, rows_v, *sems):
        gs, ws = sems[:NBUF], sems[NBUF:]
        wid = lax.axis_index("s") * nc + lax.axis_index("c")
        pltpu.sync_copy(idx_hbm.at[wid], idx_v)

        def gather(c, b):
            return pltpu.make_async_copy(
                table_hbm.at[idx_v.at[c]], rows_v.at[b], gs[b])

        def write(c, b):
            return pltpu.make_async_copy(
                rows_v.at[b], out_hbm.at[wid, c], ws[b])

        for c in range(LA):
            gather(c, c % NBUF).start()

        @pl.loop(0, n_chunks, step=NBUF)
        def _(i):
            for b in range(NBUF):
                c = i + b
                bf = (b + LA) % NBUF
                gather(c, b).wait()
                write(c, b).start()
                f = c + LA

                @pl.when(jnp.logical_and(f >= NBUF, f < n_chunks))
                def _():
                    write(f - NBUF, bf).wait()
                    gather(f, bf).start()

                if b + LA < NBUF:

                    @pl.when(f < NBUF)
                    def _():
                        gather(f, bf).start()

        for b in range(NBUF):
            write(n_chunks - NBUF + b, b).wait()

    return embed, nw, n_chunks


def kernel(token_ids, weight):
    b, s = token_ids.shape
    num_tokens = b * s
    embed, nw, n_chunks = _build(num_tokens)
    ids = token_ids.reshape(nw, n_chunks, CHUNK).astype(jnp.int32)
    out = embed(ids, weight)
    return out.reshape(b, s, EMB_DIM)


# speedup vs baseline: 5.9464x; 1.7943x over previous
import functools

import jax
import jax.numpy as jnp
from jax import lax
from jax.experimental import pallas as pl
from jax.experimental.pallas import tpu as pltpu
from jax.experimental.pallas import tpu_sc as plsc

EMB_DIM = 128
PAIR = 4
NBUF = 4
LA = 2


@functools.cache
def _build(batch: int, seq: int):
    info = plsc.get_sparse_core_info()
    nc, ns = info.num_cores, info.num_subcores
    nw = nc * ns
    assert batch % (nw * PAIR) == 0
    rows_per_w = batch // nw
    n_chunks = rows_per_w // PAIR
    assert LA < NBUF and n_chunks % NBUF == 0
    mesh = plsc.VectorSubcoreMesh(core_axis_name="c", subcore_axis_name="s")

    @functools.partial(
        pl.kernel,
        out_type=jax.ShapeDtypeStruct((batch, seq, EMB_DIM), jnp.float32),
        mesh=mesh,
        scratch_types=[
            pltpu.VMEM((rows_per_w, seq), jnp.int32),
            pltpu.VMEM((NBUF, PAIR, seq, EMB_DIM), jnp.float32),
        ]
        + [pltpu.SemaphoreType.DMA] * (2 * NBUF),
    )
    def embed(idx_hbm, table_hbm, out_hbm, idx_v, rows_v, *sems):
        gs, ws = sems[:NBUF], sems[NBUF:]
        wid = lax.axis_index("s") * nc + lax.axis_index("c")
        base = wid * rows_per_w
        pltpu.sync_copy(idx_hbm.at[pl.ds(base, rows_per_w)], idx_v)

        def gathers(c, b):
            return [
                pltpu.make_async_copy(
                    table_hbm.at[idx_v.at[c * PAIR + p]], rows_v.at[b, p], gs[b])
                for p in range(PAIR)
            ]

        def gather_start(c, b):
            for d in gathers(c, b):
                d.start()

        def gather_wait(c, b):
            for d in gathers(c, b):
                d.wait()

        def write(c, b):
            return pltpu.make_async_copy(
                rows_v.at[b], out_hbm.at[pl.ds(base + c * PAIR, PAIR)], ws[b])

        for c in range(LA):
            gather_start(c, c % NBUF)

        @pl.loop(0, n_chunks, step=NBUF)
        def _(i):
            for b in range(NBUF):
                c = i + b
                bf = (b + LA) % NBUF
                gather_wait(c, b)
                write(c, b).start()
                f = c + LA

                @pl.when(jnp.logical_and(f >= NBUF, f < n_chunks))
                def _():
                    write(f - NBUF, bf).wait()
                    gather_start(f, bf)

                if b + LA < NBUF:

                    @pl.when(f < NBUF)
                    def _():
                        gather_start(f, bf)

        for b in range(NBUF):
            write(n_chunks - NBUF + b, b).wait()

    return embed


def kernel(token_ids, weight):
    b, s = token_ids.shape
    embed = _build(b, s)
    return embed(token_ids.astype(jnp.int32), weight)


# speedup vs baseline: 10.6551x; 1.7919x over previous
import functools

import jax
import jax.numpy as jnp
from jax import lax
from jax.experimental import pallas as pl
from jax.experimental.pallas import tpu as pltpu
from jax.experimental.pallas import tpu_sc as plsc

EMB_DIM = 128
STRIPE = 128
NBUF = 5
LA = 2


@functools.cache
def _build(batch: int, seq: int):
    info = plsc.get_sparse_core_info()
    nc, ns = info.num_cores, info.num_subcores
    nw = nc * ns
    assert batch == nw * STRIPE
    n_chunks = seq
    assert LA < NBUF and n_chunks % NBUF == 0
    mesh = plsc.VectorSubcoreMesh(core_axis_name="c", subcore_axis_name="s")

    @functools.partial(
        pl.kernel,
        out_type=jax.ShapeDtypeStruct((seq, batch, EMB_DIM), jnp.float32),
        mesh=mesh,
        scratch_types=[
            pltpu.VMEM((seq, STRIPE), jnp.int32),
            pltpu.VMEM((NBUF, STRIPE, EMB_DIM), jnp.float32),
        ]
        + [pltpu.SemaphoreType.DMA] * (2 * NBUF),
    )
    def embed(idx_hbm, table_hbm, out_hbm, idx_v, rows_v, *sems):
        gs, ws = sems[:NBUF], sems[NBUF:]
        wid = lax.axis_index("s") * nc + lax.axis_index("c")
        col = wid * STRIPE
        pltpu.sync_copy(idx_hbm.at[:, pl.ds(col, STRIPE)], idx_v)

        def gather(c, b):
            return pltpu.make_async_copy(
                table_hbm.at[idx_v.at[c]], rows_v.at[b], gs[b])

        def write(c, b):
            return pltpu.make_async_copy(
                rows_v.at[b], out_hbm.at[c, pl.ds(col, STRIPE)], ws[b])

        for c in range(LA):
            gather(c, c % NBUF).start()

        @pl.loop(0, n_chunks, step=NBUF)
        def _(i):
            for b in range(NBUF):
                c = i + b
                bf = (b + LA) % NBUF
                gather(c, b).wait()
                write(c, b).start()
                f = c + LA

                @pl.when(jnp.logical_and(f >= NBUF, f < n_chunks))
                def _():
                    write(f - NBUF, bf).wait()
                    gather(f, bf).start()

                if b + LA < NBUF:

                    @pl.when(f < NBUF)
                    def _():
                        gather(f, bf).start()

        for b in range(NBUF):
            write(n_chunks - NBUF + b, b).wait()

    return embed


def kernel(token_ids, weight):
    b, s = token_ids.shape
    embed = _build(b, s)
    out_sb = embed(token_ids.T.astype(jnp.int32), weight)
    return out_sb.transpose(1, 0, 2)


# speedup vs baseline: 10.6910x; 1.0034x over previous
import functools

import jax
import jax.numpy as jnp
from jax import lax
from jax.experimental import pallas as pl
from jax.experimental.pallas import tpu as pltpu
from jax.experimental.pallas import tpu_sc as plsc

EMB_DIM = 128
STRIPE = 128
NBUF = 5
LA = 3


@functools.cache
def _build(batch: int, seq: int):
    info = plsc.get_sparse_core_info()
    nc, ns = info.num_cores, info.num_subcores
    nw = nc * ns
    assert batch == nw * STRIPE
    n_chunks = seq
    assert LA < NBUF and n_chunks % NBUF == 0
    mesh = plsc.VectorSubcoreMesh(core_axis_name="c", subcore_axis_name="s")

    @functools.partial(
        pl.kernel,
        out_type=jax.ShapeDtypeStruct((seq, batch, EMB_DIM), jnp.float32),
        mesh=mesh,
        scratch_types=[
            pltpu.VMEM((seq, STRIPE), jnp.int32),
            pltpu.VMEM((NBUF, STRIPE, EMB_DIM), jnp.float32),
        ]
        + [pltpu.SemaphoreType.DMA] * (2 * NBUF),
    )
    def embed(idx_hbm, table_hbm, out_hbm, idx_v, rows_v, *sems):
        gs, ws = sems[:NBUF], sems[NBUF:]
        wid = lax.axis_index("s") * nc + lax.axis_index("c")
        col = wid * STRIPE
        pltpu.sync_copy(idx_hbm.at[:, pl.ds(col, STRIPE)], idx_v)

        def gather(c, b):
            return pltpu.make_async_copy(
                table_hbm.at[idx_v.at[c]], rows_v.at[b], gs[b])

        def write(c, b):
            return pltpu.make_async_copy(
                rows_v.at[b], out_hbm.at[c, pl.ds(col, STRIPE)], ws[b])

        for c in range(LA):
            gather(c, c % NBUF).start()

        @pl.loop(0, n_chunks, step=NBUF)
        def _(i):
            for b in range(NBUF):
                c = i + b
                bf = (b + LA) % NBUF
                gather(c, b).wait()
                write(c, b).start()
                f = c + LA

                @pl.when(jnp.logical_and(f >= NBUF, f < n_chunks))
                def _():
                    write(f - NBUF, bf).wait()
                    gather(f, bf).start()

                if b + LA < NBUF:

                    @pl.when(f < NBUF)
                    def _():
                        gather(f, bf).start()

        for b in range(NBUF):
            write(n_chunks - NBUF + b, b).wait()

    return embed


def kernel(token_ids, weight):
    b, s = token_ids.shape
    embed = _build(b, s)
    out_sb = embed(token_ids.T.astype(jnp.int32), weight)
    return out_sb.transpose(1, 0, 2)


# speedup vs baseline: 10.7073x; 1.0015x over previous
import functools

import jax
import jax.numpy as jnp
from jax import lax
from jax.experimental import pallas as pl
from jax.experimental.pallas import tpu as pltpu
from jax.experimental.pallas import tpu_sc as plsc

EMB_DIM = 128
STRIPE = 128
HALVES = 2
SUB = STRIPE // HALVES
NBUF = 10
LA = 4


@functools.cache
def _build(batch: int, seq: int):
    info = plsc.get_sparse_core_info()
    nc, ns = info.num_cores, info.num_subcores
    nw = nc * ns
    assert batch == nw * STRIPE
    n_chunks = seq * HALVES
    assert LA < NBUF and n_chunks % NBUF == 0
    mesh = plsc.VectorSubcoreMesh(core_axis_name="c", subcore_axis_name="s")

    @functools.partial(
        pl.kernel,
        out_type=jax.ShapeDtypeStruct((seq, batch, EMB_DIM), jnp.float32),
        mesh=mesh,
        scratch_types=[
            pltpu.VMEM((seq, STRIPE), jnp.int32),
            pltpu.VMEM((NBUF, SUB, EMB_DIM), jnp.float32),
        ]
        + [pltpu.SemaphoreType.DMA] * (2 * NBUF),
    )
    def embed(idx_hbm, table_hbm, out_hbm, idx_v, rows_v, *sems):
        gs, ws = sems[:NBUF], sems[NBUF:]
        wid = lax.axis_index("s") * nc + lax.axis_index("c")
        col = wid * STRIPE
        pltpu.sync_copy(idx_hbm.at[:, pl.ds(col, STRIPE)], idx_v)

        def gather(c, b):
            s, off = c // HALVES, (c % HALVES) * SUB
            return pltpu.make_async_copy(
                table_hbm.at[idx_v.at[s, pl.ds(off, SUB)]], rows_v.at[b], gs[b])

        def write(c, b):
            s, off = c // HALVES, (c % HALVES) * SUB
            return pltpu.make_async_copy(
                rows_v.at[b], out_hbm.at[s, pl.ds(col + off, SUB)], ws[b])

        for c in range(LA):
            gather(c, c % NBUF).start()

        @pl.loop(0, n_chunks, step=NBUF)
        def _(i):
            for b in range(NBUF):
                c = i + b
                bf = (b + LA) % NBUF
                gather(c, b).wait()
                write(c, b).start()
                f = c + LA

                @pl.when(jnp.logical_and(f >= NBUF, f < n_chunks))
                def _():
                    write(f - NBUF, bf).wait()
                    gather(f, bf).start()

                if b + LA < NBUF:

                    @pl.when(f < NBUF)
                    def _():
                        gather(f, bf).start()

        for b in range(NBUF):
            write(n_chunks - NBUF + b, b).wait()

    return embed


def kernel(token_ids, weight):
    b, s = token_ids.shape
    embed = _build(b, s)
    out_sb = embed(token_ids.T.astype(jnp.int32), weight)
    return out_sb.transpose(1, 0, 2)


# speedup vs baseline: 10.7663x; 1.0055x over previous
import functools

import jax
import jax.numpy as jnp
from jax import lax
from jax.experimental import pallas as pl
from jax.experimental.pallas import tpu as pltpu
from jax.experimental.pallas import tpu_sc as plsc

EMB_DIM = 128
STRIPE = 128
HALVES = 2
SUB = STRIPE // HALVES
NBUF = 10
LA = 6


@functools.cache
def _build(batch: int, seq: int):
    info = plsc.get_sparse_core_info()
    nc, ns = info.num_cores, info.num_subcores
    nw = nc * ns
    assert batch == nw * STRIPE
    n_chunks = seq * HALVES
    assert LA < NBUF and n_chunks % NBUF == 0
    mesh = plsc.VectorSubcoreMesh(core_axis_name="c", subcore_axis_name="s")

    @functools.partial(
        pl.kernel,
        out_type=jax.ShapeDtypeStruct((seq, batch, EMB_DIM), jnp.float32),
        mesh=mesh,
        scratch_types=[
            pltpu.VMEM((seq, STRIPE), jnp.int32),
            pltpu.VMEM((NBUF, SUB, EMB_DIM), jnp.float32),
        ]
        + [pltpu.SemaphoreType.DMA] * (2 * NBUF),
    )
    def embed(idx_hbm, table_hbm, out_hbm, idx_v, rows_v, *sems):
        gs, ws = sems[:NBUF], sems[NBUF:]
        wid = lax.axis_index("s") * nc + lax.axis_index("c")
        col = wid * STRIPE
        pltpu.sync_copy(idx_hbm.at[:, pl.ds(col, STRIPE)], idx_v)

        def gather(c, b):
            s, off = c // HALVES, (c % HALVES) * SUB
            return pltpu.make_async_copy(
                table_hbm.at[idx_v.at[s, pl.ds(off, SUB)]], rows_v.at[b], gs[b])

        def write(c, b):
            s, off = c // HALVES, (c % HALVES) * SUB
            return pltpu.make_async_copy(
                rows_v.at[b], out_hbm.at[s, pl.ds(col + off, SUB)], ws[b])

        for c in range(LA):
            gather(c, c % NBUF).start()

        @pl.loop(0, n_chunks, step=NBUF)
        def _(i):
            for b in range(NBUF):
                c = i + b
                bf = (b + LA) % NBUF
                gather(c, b).wait()
                write(c, b).start()
                f = c + LA

                @pl.when(jnp.logical_and(f >= NBUF, f < n_chunks))
                def _():
                    write(f - NBUF, bf).wait()
                    gather(f, bf).start()

                if b + LA < NBUF:

                    @pl.when(f < NBUF)
                    def _():
                        gather(f, bf).start()

        for b in range(NBUF):
            write(n_chunks - NBUF + b, b).wait()

    return embed


def kernel(token_ids, weight):
    b, s = token_ids.shape
    embed = _build(b, s)
    out_sb = embed(token_ids.T.astype(jnp.int32), weight)
    return out_sb.transpose(1, 0, 2)
